# baseline (speedup 1.0000x reference)
import jax
import jax.numpy as jnp
from jax import lax
from jax.experimental import pallas as pl
from jax.experimental.pallas import tpu as pltpu
from jax.experimental.pallas import tpu_sc as plsc

_SIZE = 1.0
_RES = 64
_EMBED = 16
_TABLE = 2 ** 21
_N = 1048576

_NC = 2
_NS = 16
_NW = _NC * _NS
_L = 16

_PER_W = _N // _NW
_CHUNK = 2048
_NCHUNK = _PER_W // _CHUNK
_GSUB = 128
_NG = _CHUNK // _GSUB

_P2 = 2654435761
_P3 = 805459861


def _tec_body(xyz_hbm, table_hbm, out_hbm, xyz_v, idx_v, rows_v, gsem):
    wid = lax.axis_index("s") * _NC + lax.axis_index("c")
    base = wid * _PER_W

    lane3 = lax.iota(jnp.int32, _L) * 3
    p2 = jnp.full((_L,), jnp.int32(_P2 - 2 ** 32), jnp.int32)
    p3 = jnp.full((_L,), jnp.int32(_P3), jnp.int32)
    mask = jnp.full((_L,), _TABLE - 1, jnp.int32)

    def chunk_body(c, carry):
        pstart = base + c * _CHUNK
        pltpu.sync_copy(xyz_hbm.at[pl.ds(pstart * 3, _CHUNK * 3)], xyz_v)

        def hash_body(i, carry2):
            fbase = lane3 + i * (3 * _L)
            x = plsc.load_gather(xyz_v, [fbase])
            y = plsc.load_gather(xyz_v, [fbase + 1])
            z = plsc.load_gather(xyz_v, [fbase + 2])

            def quant(v):
                f = (v / _SIZE + 0.5) * _RES
                return jnp.minimum(f.astype(jnp.int32), _RES - 1)

            cx, cy, cz = quant(x), quant(y), quant(z)
            h = cx ^ (cy * p2) ^ (cz * p3)
            idx_v[pl.ds(i * _L, _L)] = h & mask
            return carry2

        lax.fori_loop(0, _CHUNK // _L, hash_body, 0)

        descs = []
        for g in range(_NG):
            descs.append(
                pltpu.async_copy(
                    table_hbm.at[idx_v.at[pl.ds(g * _GSUB, _GSUB)]],
                    rows_v.at[pl.ds(g * _GSUB, _GSUB)],
                    gsem,
                )
            )
        for d in descs:
            d.wait()

        pltpu.sync_copy(rows_v, out_hbm.at[pl.ds(pstart, _CHUNK)])
        return carry

    lax.fori_loop(0, _NCHUNK, chunk_body, 0)


@jax.jit
def _lookup(xyz, table):
    mesh = plsc.VectorSubcoreMesh(
        core_axis_name="c", subcore_axis_name="s",
        num_cores=_NC, num_subcores=_NS,
    )
    run = pl.kernel(
        _tec_body,
        out_type=jax.ShapeDtypeStruct((_N, _EMBED), jnp.float32),
        mesh=mesh,
        scratch_types=[
            pltpu.VMEM((_CHUNK * 3,), jnp.float32),
            pltpu.VMEM((_CHUNK,), jnp.int32),
            pltpu.VMEM((_CHUNK, _EMBED), jnp.float32),
            pltpu.SemaphoreType.DMA,
        ],
        compiler_params=pltpu.CompilerParams(
            needs_layout_passes=False, use_tc_tiling_on_sc=False),
    )
    return run(xyz.reshape(-1), table)


def kernel(xyz, table):
    return _lookup(xyz, table)


# speedup vs baseline: 1.9986x; 1.9986x over previous
import jax
import jax.numpy as jnp
from jax import lax
from jax.experimental import pallas as pl
from jax.experimental.pallas import tpu as pltpu
from jax.experimental.pallas import tpu_sc as plsc

_SIZE = 1.0
_RES = 64
_EMBED = 16
_TABLE = 2 ** 21
_N = 1048576

_NC = 2
_NS = 16
_NW = _NC * _NS
_L = 16

_PER_W = _N // _NW
_CHUNK = 2048
_NCHUNK = _PER_W // _CHUNK
_GSUB = 128
_NG = _CHUNK // _GSUB

_P2 = 2654435761
_P3 = 805459861


def _hash_body(x_ref, y_ref, z_ref, o_ref):
    def quant(v):
        f = (v / _SIZE + 0.5) * _RES
        return jnp.minimum(f.astype(jnp.int32), _RES - 1)

    cx = quant(x_ref[...])
    cy = quant(y_ref[...])
    cz = quant(z_ref[...])
    h = cx ^ (cy * jnp.int32(_P2 - 2 ** 32)) ^ (cz * jnp.int32(_P3))
    o_ref[...] = h & jnp.int32(_TABLE - 1)


def _gather_body(idx_hbm, table_hbm, out_hbm, idx_v, rows_v, gsem):
    wid = lax.axis_index("s") * _NC + lax.axis_index("c")
    base = wid * _PER_W

    def chunk_body(c, carry):
        pstart = base + c * _CHUNK
        pltpu.sync_copy(idx_hbm.at[pl.ds(pstart, _CHUNK)], idx_v)
        descs = []
        for g in range(_NG):
            descs.append(
                pltpu.async_copy(
                    table_hbm.at[idx_v.at[pl.ds(g * _GSUB, _GSUB)]],
                    rows_v.at[pl.ds(g * _GSUB, _GSUB)],
                    gsem,
                )
            )
        for d in descs:
            d.wait()
        pltpu.sync_copy(rows_v, out_hbm.at[pl.ds(pstart, _CHUNK)])
        return carry

    lax.fori_loop(0, _NCHUNK, chunk_body, 0)


@jax.jit
def _lookup(xyz, table):
    x = xyz[:, 0].reshape(_N // 128, 128)
    y = xyz[:, 1].reshape(_N // 128, 128)
    z = xyz[:, 2].reshape(_N // 128, 128)

    idx2d = pl.pallas_call(
        _hash_body,
        out_shape=jax.ShapeDtypeStruct((_N // 128, 128), jnp.int32),
    )(x, y, z)
    idx_flat = idx2d.reshape(-1)

    t_flat = lax.optimization_barrier(table.reshape(-1))
    t2 = t_flat.reshape(_TABLE, _EMBED)

    mesh = plsc.VectorSubcoreMesh(
        core_axis_name="c", subcore_axis_name="s",
        num_cores=_NC, num_subcores=_NS,
    )
    run = pl.kernel(
        _gather_body,
        out_type=jax.ShapeDtypeStruct((_N, _EMBED), jnp.float32),
        mesh=mesh,
        scratch_types=[
            pltpu.VMEM((_CHUNK,), jnp.int32),
            pltpu.VMEM((_CHUNK, _EMBED), jnp.float32),
            pltpu.SemaphoreType.DMA,
        ],
        compiler_params=pltpu.CompilerParams(
            needs_layout_passes=False, use_tc_tiling_on_sc=False),
    )
    out2d = run(idx_flat, t2)

    out_flat = lax.optimization_barrier(out2d.reshape(-1))
    return out_flat.reshape(_N, _EMBED)


def kernel(xyz, table):
    return _lookup(xyz, table)


# speedup vs baseline: 2.6339x; 1.3179x over previous
import jax
import jax.numpy as jnp
from jax import lax
from jax.experimental import pallas as pl
from jax.experimental.pallas import tpu as pltpu
from jax.experimental.pallas import tpu_sc as plsc

_SIZE = 1.0
_RES = 64
_EMBED = 16
_TABLE = 2 ** 21
_N = 1048576

_NC = 2
_NS = 16
_NW = _NC * _NS
_L = 16

_PER_W = _N // _NW
_CHUNK = 2048
_NCHUNK = _PER_W // _CHUNK
_GSUB = 128
_NG = _CHUNK // _GSUB

_NBLK = _TABLE // 128
_BLK_W = _NBLK // _NW
_RB = 16
_RCH = _BLK_W // _RB
_RENT = _RB * 128

_P2 = 2654435761
_P3 = 805459861


def _hash_tc_body(x_ref, y_ref, z_ref, o_ref):
    def quant(v):
        f = (v / _SIZE + 0.5) * _RES
        return jnp.minimum(f.astype(jnp.int32), _RES - 1)

    cx = quant(x_ref[...])
    cy = quant(y_ref[...])
    cz = quant(z_ref[...])
    h = cx ^ (cy * jnp.int32(_P2 - 2 ** 32)) ^ (cz * jnp.int32(_P3))
    o_ref[...] = h & jnp.int32(_TABLE - 1)


def _relayout_body(nt_hbm, out_hbm, buf_v, rows_v):
    wid = lax.axis_index("s") * _NC + lax.axis_index("c")
    lane = lax.iota(jnp.int32, _L)

    def chunk_body(c, carry):
        b0 = wid * _BLK_W + c * _RB
        for e_half in range(2):
            pltpu.sync_copy(nt_hbm.at[e_half, pl.ds(b0, _RB)],
                            buf_v.at[e_half])

        def b_body(b, carry2):
            def f_body(fl, carry3):
                for e_half in range(2):
                    f = 8 * e_half + fl
                    coli = jnp.full((_L,), 0, jnp.int32) + f
                    for g in range(8):
                        vals = buf_v[e_half, b, fl, pl.ds(g * _L, _L)]
                        rowi = lane + (128 * b + g * _L)
                        plsc.store_scatter(rows_v, [rowi, coli], vals)
                return carry3
            lax.fori_loop(0, 8, f_body, 0)
            return carry2
        lax.fori_loop(0, _RB, b_body, 0)

        pltpu.sync_copy(rows_v,
                        out_hbm.at[pl.ds(wid * _BLK_W * 128 + c * _RENT,
                                         _RENT)])
        return carry

    lax.fori_loop(0, _RCH, chunk_body, 0)


def _gather_body(idx_hbm, table_hbm, out_hbm, idx_v, rows_v, gsem):
    wid = lax.axis_index("s") * _NC + lax.axis_index("c")
    base = wid * _PER_W

    def chunk_body(c, carry):
        pstart = base + c * _CHUNK
        pltpu.sync_copy(idx_hbm.at[pl.ds(pstart, _CHUNK)], idx_v)
        descs = []
        for g in range(_NG):
            descs.append(
                pltpu.async_copy(
                    table_hbm.at[idx_v.at[pl.ds(g * _GSUB, _GSUB)]],
                    rows_v.at[pl.ds(g * _GSUB, _GSUB)],
                    gsem,
                )
            )
        for d in descs:
            d.wait()
        pltpu.sync_copy(rows_v, out_hbm.at[pl.ds(pstart, _CHUNK)])
        return carry

    lax.fori_loop(0, _NCHUNK, chunk_body, 0)


_SC_PARAMS = pltpu.CompilerParams(
    needs_layout_passes=False, use_tc_tiling_on_sc=False)


@jax.jit
def _lookup(xyz, table):
    x = xyz[:, 0].reshape(_N // 128, 128)
    y = xyz[:, 1].reshape(_N // 128, 128)
    z = xyz[:, 2].reshape(_N // 128, 128)

    idx2d = pl.pallas_call(
        _hash_tc_body,
        out_shape=jax.ShapeDtypeStruct((_N // 128, 128), jnp.int32),
    )(x, y, z)
    idx_flat = idx2d.reshape(-1)

    nt = table.T.reshape(2, 8, _NBLK, 128).transpose(0, 2, 1, 3)

    mesh = plsc.VectorSubcoreMesh(
        core_axis_name="c", subcore_axis_name="s",
        num_cores=_NC, num_subcores=_NS,
    )
    table_rm = pl.kernel(
        _relayout_body,
        out_type=jax.ShapeDtypeStruct((_TABLE, _EMBED), jnp.float32),
        mesh=mesh,
        scratch_types=[
            pltpu.VMEM((2, _RB, 8, 128), jnp.float32),
            pltpu.VMEM((_RENT, _EMBED), jnp.float32),
        ],
        compiler_params=_SC_PARAMS,
    )(nt)

    out2d = pl.kernel(
        _gather_body,
        out_type=jax.ShapeDtypeStruct((_N, _EMBED), jnp.float32),
        mesh=mesh,
        scratch_types=[
            pltpu.VMEM((_CHUNK,), jnp.int32),
            pltpu.VMEM((_CHUNK, _EMBED), jnp.float32),
            pltpu.SemaphoreType.DMA,
        ],
        compiler_params=_SC_PARAMS,
    )(idx_flat, table_rm)

    out_flat = lax.optimization_barrier(out2d.reshape(-1))
    return out_flat.reshape(_N, _EMBED)


def kernel(xyz, table):
    return _lookup(xyz, table)


# speedup vs baseline: 3.8444x; 1.4596x over previous
import jax
import jax.numpy as jnp
from jax import lax
from jax.experimental import pallas as pl
from jax.experimental.pallas import tpu as pltpu
from jax.experimental.pallas import tpu_sc as plsc

_SIZE = 1.0
_RES = 64
_EMBED = 16
_TABLE = 2 ** 21
_N = 1048576

_NC = 2
_NS = 16
_NW = _NC * _NS
_L = 16

_PER_W = _N // _NW
_CHUNK = 2048
_NCHUNK = _PER_W // _CHUNK
_GSUB = 128
_NG = _CHUNK // _GSUB
_OBLK_W = _PER_W // 128

_NBLK = _TABLE // 128
_BLK_W = _NBLK // _NW
_RB = 8
_RCH = _BLK_W // _RB
_RENT = _RB * 128

_P2 = 2654435761
_P3 = 805459861


def _hash_tc_body(x_ref, y_ref, z_ref, o_ref):
    def quant(v):
        f = (v / _SIZE + 0.5) * _RES
        return jnp.minimum(f.astype(jnp.int32), _RES - 1)

    cx = quant(x_ref[...])
    cy = quant(y_ref[...])
    cz = quant(z_ref[...])
    h = cx ^ (cy * jnp.int32(_P2 - 2 ** 32)) ^ (cz * jnp.int32(_P3))
    o_ref[...] = h & jnp.int32(_TABLE - 1)


def _relayout_body(nt_hbm, out_hbm, buf0, buf1, rows0, rows1,
                   sem_in, sem_out):
    wid = lax.axis_index("s") * _NC + lax.axis_index("c")
    lane16 = lax.iota(jnp.int32, _L) * 16
    bufs = (buf0, buf1)
    rows = (rows0, rows1)

    def in_descs(par, c):
        b0 = wid * _BLK_W + c * _RB
        return [
            pltpu.make_async_copy(nt_hbm.at[e, pl.ds(b0, _RB)],
                                  bufs[par].at[e], sem_in)
            for e in range(2)
        ]

    def out_desc(par, c):
        ent0 = (wid * _BLK_W + c * _RB) * 128
        return pltpu.make_async_copy(
            rows[par], out_hbm.at[pl.ds(ent0 * _EMBED, _RENT * _EMBED)],
            sem_out)

    for d in in_descs(0, 0):
        d.start()

    @pl.loop(0, _RCH, step=2)
    def outer(i):
        for par in range(2):
            c = i + par
            for d in in_descs(par, c):
                d.wait()

            @pl.when(c + 1 < _RCH)
            def _():
                for d in in_descs(1 - par, c + 1):
                    d.start()

            @pl.when(c >= 2)
            def _():
                out_desc(par, c - 2).wait()

            def blk_body(blk, carry):
                vb = lane16 + blk * (128 * _EMBED)
                for e in range(2):
                    for fl in range(8):
                        f = 8 * e + fl
                        for g in range(8):
                            vals = bufs[par][e, blk, fl, pl.ds(g * _L, _L)]
                            plsc.store_scatter(
                                rows[par], [vb + (g * _L * _EMBED + f)], vals)
                return carry

            lax.fori_loop(0, _RB, blk_body, 0)
            out_desc(par, c).start()

    for c in (_RCH - 2, _RCH - 1):
        out_desc(c % 2, c).wait()


def _gather_body(idx_hbm, table_hbm, out_hbm, idx_v, rows_v, tr_v, gsem):
    wid = lax.axis_index("s") * _NC + lax.axis_index("c")
    base = wid * _PER_W
    lane = lax.iota(jnp.int32, _L)
    colf = [jnp.full((_L,), f, jnp.int32) for f in range(_EMBED)]

    def chunk_body(c, carry):
        pstart = base + c * _CHUNK
        pltpu.sync_copy(idx_hbm.at[pl.ds(pstart, _CHUNK)], idx_v)
        descs = []
        for g in range(_NG):
            descs.append(
                pltpu.async_copy(
                    table_hbm.at[idx_v.at[pl.ds(g * _GSUB, _GSUB)]],
                    rows_v.at[pl.ds(g * _GSUB, _GSUB)],
                    gsem,
                )
            )
        for d in descs:
            d.wait()

        def blk_body(blk, carry2):
            vb = lane + blk * 128
            for e in range(2):
                for fl in range(8):
                    f = 8 * e + fl
                    for g in range(8):
                        vals = plsc.load_gather(
                            rows_v, [vb + g * _L, colf[f]])
                        tr_v[e, blk, fl, pl.ds(g * _L, _L)] = vals
            return carry2

        lax.fori_loop(0, _CHUNK // 128, blk_body, 0)

        blk0 = wid * _OBLK_W + c * (_CHUNK // 128)
        for e in range(2):
            pltpu.sync_copy(tr_v.at[e],
                            out_hbm.at[e, pl.ds(blk0, _CHUNK // 128)])
        return carry

    lax.fori_loop(0, _NCHUNK, chunk_body, 0)


_SC_PARAMS = pltpu.CompilerParams(
    needs_layout_passes=False, use_tc_tiling_on_sc=False)


@jax.jit
def _lookup(xyz, table):
    x = xyz[:, 0].reshape(_N // 128, 128)
    y = xyz[:, 1].reshape(_N // 128, 128)
    z = xyz[:, 2].reshape(_N // 128, 128)

    idx2d = pl.pallas_call(
        _hash_tc_body,
        out_shape=jax.ShapeDtypeStruct((_N // 128, 128), jnp.int32),
    )(x, y, z)
    idx_flat = idx2d.reshape(-1)

    nt = table.T.reshape(2, 8, _NBLK, 128).transpose(0, 2, 1, 3)

    mesh = plsc.VectorSubcoreMesh(
        core_axis_name="c", subcore_axis_name="s",
        num_cores=_NC, num_subcores=_NS,
    )
    table_rm_flat = pl.kernel(
        _relayout_body,
        out_type=jax.ShapeDtypeStruct((_TABLE * _EMBED,), jnp.float32),
        mesh=mesh,
        scratch_types=[
            pltpu.VMEM((2, _RB, 8, 128), jnp.float32),
            pltpu.VMEM((2, _RB, 8, 128), jnp.float32),
            pltpu.VMEM((_RENT * _EMBED,), jnp.float32),
            pltpu.VMEM((_RENT * _EMBED,), jnp.float32),
            pltpu.SemaphoreType.DMA,
            pltpu.SemaphoreType.DMA,
        ],
        compiler_params=_SC_PARAMS,
    )(nt)
    table_rm = table_rm_flat.reshape(_TABLE, _EMBED)

    out4 = pl.kernel(
        _gather_body,
        out_type=jax.ShapeDtypeStruct((2, _N // 128, 8, 128), jnp.float32),
        mesh=mesh,
        scratch_types=[
            pltpu.VMEM((_CHUNK,), jnp.int32),
            pltpu.VMEM((_CHUNK, _EMBED), jnp.float32),
            pltpu.VMEM((2, _CHUNK // 128, 8, 128), jnp.float32),
            pltpu.SemaphoreType.DMA,
        ],
        compiler_params=_SC_PARAMS,
    )(idx_flat, table_rm)

    return out4.transpose(1, 3, 0, 2).reshape(_N, _EMBED)


def kernel(xyz, table):
    return _lookup(xyz, table)
